# baseline (speedup 1.0000x reference)
import functools

import jax
import jax.numpy as jnp
from jax import lax
from jax.experimental import pallas as pl
from jax.experimental.pallas import tpu as pltpu
from jax.experimental.pallas import tpu_sc as plsc

N_ROWS = 50000
FEAT = 512
NSEG = 64
LANES = 16
NGRP = FEAT // LANES

NC = 2
NS = 16
NW = NC * NS

P = 80
NPIECES = N_ROWS // P


def _sc_segsum_body(x_hbm, batch_hbm, parts_out, cnts_out,
                    x_buf0, x_buf1, idx_buf0, idx_buf1, acc, cnt,
                    sem0, sem0b, sem1, sem1b):
    cid = lax.axis_index("c")
    sid = lax.axis_index("s")
    g = cid * NS + sid

    base0 = g * P
    pltpu.async_copy(x_hbm.at[pl.ds(base0, 48)], x_buf0.at[pl.ds(0, 48)],
                     sem0)
    pltpu.async_copy(batch_hbm.at[pl.ds(base0, P)], idx_buf0, sem0)
    pltpu.async_copy(x_hbm.at[pl.ds(base0 + 48, P - 48)],
                     x_buf0.at[pl.ds(48, P - 48)], sem0b)

    def _zero_row(i, _):
        def _zero_grp(c, _):
            acc[i, pl.ds(c * LANES, LANES)] = jnp.zeros((LANES,), jnp.float32)
            return 0
        lax.fori_loop(0, NGRP, _zero_grp, 0)
        cnt[i, :] = jnp.zeros((LANES,), jnp.float32)
        return 0
    lax.fori_loop(0, NSEG, _zero_row, 0)

    ones = jnp.ones((LANES,), jnp.float32)
    sixteens = jnp.full((LANES,), float(LANES), jnp.float32)

    def _accum_rows(x_buf, idx_buf, j0, j1):
        def _row16(j, _):
            segv = idx_buf[pl.ds(j * LANES, LANES)]
            seg0 = segv[0]
            segF = segv[LANES - 1]
            base_r = j * LANES

            @pl.when(seg0 == segF)
            def _fast():
                def _grp(c, _):
                    off = c * LANES
                    vals = [x_buf[base_r + k, pl.ds(off, LANES)]
                            for k in range(LANES)]
                    while len(vals) > 1:
                        vals = [vals[i] + vals[i + 1]
                                for i in range(0, len(vals), 2)]
                    plsc.addupdate(acc.at[seg0, pl.ds(off, LANES)], vals[0])
                    return 0
                lax.fori_loop(0, NGRP, _grp, 0)
                plsc.addupdate(cnt.at[seg0], sixteens)

            @pl.when(seg0 != segF)
            def _slow():
                for k in range(LANES):
                    seg = segv[k]
                    vals = [x_buf[base_r + k, pl.ds(c * LANES, LANES)]
                            for c in range(NGRP)]
                    for c in range(NGRP):
                        plsc.addupdate(acc.at[seg, pl.ds(c * LANES, LANES)],
                                       vals[c])
                    plsc.addupdate(cnt.at[seg], ones)
            return 0
        lax.fori_loop(j0, j1, _row16, 0)

    n_mine = (NPIECES - g + NW - 1) // NW

    H1 = 48
    H2 = P - H1

    bufs = ((x_buf0, idx_buf0, sem0, sem0b), (x_buf1, idx_buf1, sem1, sem1b))

    def _issue(k, b):
        base = (g + k * NW) * P
        x_b, i_b, s_a, s_b2 = bufs[b]
        pltpu.async_copy(x_hbm.at[pl.ds(base, H1)], x_b.at[pl.ds(0, H1)], s_a)
        pltpu.async_copy(batch_hbm.at[pl.ds(base, P)], i_b, s_a)
        pltpu.async_copy(x_hbm.at[pl.ds(base + H1, H2)],
                         x_b.at[pl.ds(H1, H2)], s_b2)

    def _wait_and_accum(k, b):
        base = (g + k * NW) * P
        x_b, i_b, s_a, s_b2 = bufs[b]
        pltpu.make_async_copy(x_hbm.at[pl.ds(base, H1)],
                              x_b.at[pl.ds(0, H1)], s_a).wait()
        pltpu.make_async_copy(batch_hbm.at[pl.ds(base, P)], i_b, s_a).wait()
        _accum_rows(x_b, i_b, 0, H1 // LANES)
        pltpu.make_async_copy(x_hbm.at[pl.ds(base + H1, H2)],
                              x_b.at[pl.ds(H1, H2)], s_b2).wait()
        _accum_rows(x_b, i_b, H1 // LANES, P // LANES)

    def _piece(k, _):
        even = lax.rem(k, 2) == 0

        @pl.when(k + 1 < n_mine)
        def _prefetch():
            @pl.when(even)
            def _():
                _issue(k + 1, 1)

            @pl.when(jnp.logical_not(even))
            def _():
                _issue(k + 1, 0)

        @pl.when(even)
        def _():
            _wait_and_accum(k, 0)

        @pl.when(jnp.logical_not(even))
        def _():
            _wait_and_accum(k, 1)
        return 0
    lax.fori_loop(0, n_mine, _piece, 0)

    pltpu.sync_copy(acc, parts_out.at[g])
    pltpu.sync_copy(cnt, cnts_out.at[g])


_sc_segsum = functools.partial(
    pl.kernel,
    out_type=(
        jax.ShapeDtypeStruct((NW, NSEG, FEAT), jnp.float32),
        jax.ShapeDtypeStruct((NW, NSEG, LANES), jnp.float32),
    ),
    mesh=plsc.VectorSubcoreMesh(core_axis_name="c", subcore_axis_name="s"),
    scratch_types=(
        pltpu.VMEM((P, FEAT), jnp.float32),
        pltpu.VMEM((P, FEAT), jnp.float32),
        pltpu.VMEM((P,), jnp.int32),
        pltpu.VMEM((P,), jnp.int32),
        pltpu.VMEM((NSEG, FEAT), jnp.float32),
        pltpu.VMEM((NSEG, LANES), jnp.float32),
        pltpu.SemaphoreType.DMA,
        pltpu.SemaphoreType.DMA,
        pltpu.SemaphoreType.DMA,
        pltpu.SemaphoreType.DMA,
    ),
)(_sc_segsum_body)


BLK = 5000
NBLK = N_ROWS // BLK


def _main_body(parts_ref, cnts_ref, wt_ref, b_ref, x_ref, seg_ref, out_ref,
               wct_s, corr_s):
    @pl.when(pl.program_id(0) == 0)
    def _prep():
        sums = jnp.sum(parts_ref[...], axis=0)
        cnt = jnp.sum(cnts_ref[...], axis=0)
        cnt0 = cnt[:, 0:1]
        mean = sums / jnp.maximum(cnt0, 1.0)
        w1t = wt_ref[0:FEAT, :]
        w2t = wt_ref[FEAT:2 * FEAT, :]
        corr = b_ref[...] - jnp.dot(mean, w2t,
                                    preferred_element_type=jnp.float32)
        corr_s[...] = corr.astype(jnp.bfloat16)
        wct_s[...] = (w1t + w2t).astype(jnp.bfloat16)

    xb = x_ref[...].astype(jnp.bfloat16)
    acc = jnp.dot(xb, wct_s[...], preferred_element_type=jnp.float32)
    seg = seg_ref[...].reshape(1, BLK)
    onehot = (lax.broadcasted_iota(jnp.int32, (NSEG, BLK), 0)
              == jnp.broadcast_to(seg, (NSEG, BLK))).astype(jnp.bfloat16)
    acc = acc + lax.dot_general(onehot, corr_s[...],
                                (((0,), (0,)), ((), ())),
                                preferred_element_type=jnp.float32)
    out_ref[...] = acc


def _main(parts, cnts, wt, b2, x, seg3):
    return pl.pallas_call(
        _main_body,
        grid=(NBLK,),
        in_specs=[
            pl.BlockSpec((NW, NSEG, FEAT), lambda i: (0, 0, 0)),
            pl.BlockSpec((NW, NSEG, LANES), lambda i: (0, 0, 0)),
            pl.BlockSpec((2 * FEAT, FEAT), lambda i: (0, 0)),
            pl.BlockSpec((1, FEAT), lambda i: (0, 0)),
            pl.BlockSpec((BLK, FEAT), lambda i: (i, 0)),
            pl.BlockSpec((1, 1, BLK), lambda i: (i, 0, 0)),
        ],
        out_specs=pl.BlockSpec((BLK, FEAT), lambda i: (i, 0)),
        out_shape=jax.ShapeDtypeStruct((N_ROWS, FEAT), jnp.float32),
        scratch_shapes=[
            pltpu.VMEM((FEAT, FEAT), jnp.bfloat16),
            pltpu.VMEM((NSEG, FEAT), jnp.bfloat16),
        ],
    )(parts, cnts, wt, b2, x, seg3)


def kernel(x, batch, W, b):
    batch32 = batch.astype(jnp.int32)
    parts, cnts = _sc_segsum(x, batch32)
    wt = W.T
    b2 = b.reshape(1, FEAT)
    seg3 = batch32.reshape(NBLK, 1, BLK)
    return _main(parts, cnts, wt, b2, x, seg3)


# speedup vs baseline: 1.0309x; 1.0309x over previous
import functools

import jax
import jax.numpy as jnp
from jax import lax
from jax.experimental import pallas as pl
from jax.experimental.pallas import tpu as pltpu
from jax.experimental.pallas import tpu_sc as plsc

N_ROWS = 50000
FEAT = 512
NSEG = 64
LANES = 16
NGRP = FEAT // LANES

NC = 2
NS = 16
NW = NC * NS

P = 80
NPIECES = N_ROWS // P


def _sc_segsum_body(x_hbm, batch_hbm, parts_out, cnts_out,
                    x_buf0, x_buf1, idx_buf0, idx_buf1, acc, cnt,
                    sem0, sem1):
    cid = lax.axis_index("c")
    sid = lax.axis_index("s")
    g = cid * NS + sid

    base0 = g * P
    pltpu.async_copy(x_hbm.at[pl.ds(base0, P)], x_buf0, sem0)
    pltpu.async_copy(batch_hbm.at[pl.ds(base0, P)], idx_buf0, sem0)

    def _zero_row(i, _):
        def _zero_grp(c, _):
            acc[i, pl.ds(c * LANES, LANES)] = jnp.zeros((LANES,), jnp.float32)
            return 0
        lax.fori_loop(0, NGRP, _zero_grp, 0)
        cnt[i, :] = jnp.zeros((LANES,), jnp.float32)
        return 0
    lax.fori_loop(0, NSEG, _zero_row, 0)

    ones = jnp.ones((LANES,), jnp.float32)
    sixteens = jnp.full((LANES,), float(LANES), jnp.float32)

    def _accum_rows(x_buf, idx_buf, nrows16):
        def _row16(j, _):
            segv = idx_buf[pl.ds(j * LANES, LANES)]
            seg0 = segv[0]
            segF = segv[LANES - 1]
            base_r = j * LANES

            @pl.when(seg0 == segF)
            def _fast():
                def _grp(c, _):
                    off = c * LANES
                    vals = [x_buf[base_r + k, pl.ds(off, LANES)]
                            for k in range(LANES)]
                    while len(vals) > 1:
                        vals = [vals[i] + vals[i + 1]
                                for i in range(0, len(vals), 2)]
                    plsc.addupdate(acc.at[seg0, pl.ds(off, LANES)], vals[0])
                    return 0
                lax.fori_loop(0, NGRP, _grp, 0)
                plsc.addupdate(cnt.at[seg0], sixteens)

            @pl.when(seg0 != segF)
            def _slow():
                for k in range(LANES):
                    seg = segv[k]
                    vals = [x_buf[base_r + k, pl.ds(c * LANES, LANES)]
                            for c in range(NGRP)]
                    for c in range(NGRP):
                        plsc.addupdate(acc.at[seg, pl.ds(c * LANES, LANES)],
                                       vals[c])
                    plsc.addupdate(cnt.at[seg], ones)
            return 0
        lax.fori_loop(0, nrows16, _row16, 0)

    n_mine = (NPIECES - g + NW - 1) // NW

    bufs = ((x_buf0, idx_buf0, sem0), (x_buf1, idx_buf1, sem1))

    def _issue(k, b):
        base = (g + k * NW) * P
        x_b, i_b, s_b = bufs[b]
        pltpu.async_copy(x_hbm.at[pl.ds(base, P)], x_b, s_b)
        pltpu.async_copy(batch_hbm.at[pl.ds(base, P)], i_b, s_b)

    def _wait_and_accum(k, b):
        base = (g + k * NW) * P
        x_b, i_b, s_b = bufs[b]
        pltpu.make_async_copy(x_hbm.at[pl.ds(base, P)], x_b, s_b).wait()
        pltpu.make_async_copy(batch_hbm.at[pl.ds(base, P)], i_b, s_b).wait()
        _accum_rows(x_b, i_b, P // LANES)

    def _piece(k, _):
        even = lax.rem(k, 2) == 0

        @pl.when(k + 1 < n_mine)
        def _prefetch():
            @pl.when(even)
            def _():
                _issue(k + 1, 1)

            @pl.when(jnp.logical_not(even))
            def _():
                _issue(k + 1, 0)

        @pl.when(even)
        def _():
            _wait_and_accum(k, 0)

        @pl.when(jnp.logical_not(even))
        def _():
            _wait_and_accum(k, 1)
        return 0
    lax.fori_loop(0, n_mine, _piece, 0)

    pltpu.sync_copy(acc, parts_out.at[g])
    pltpu.sync_copy(cnt, cnts_out.at[g])


_sc_segsum = functools.partial(
    pl.kernel,
    out_type=(
        jax.ShapeDtypeStruct((NW, NSEG, FEAT), jnp.float32),
        jax.ShapeDtypeStruct((NW, NSEG, LANES), jnp.float32),
    ),
    mesh=plsc.VectorSubcoreMesh(core_axis_name="c", subcore_axis_name="s"),
    scratch_types=(
        pltpu.VMEM((P, FEAT), jnp.float32),
        pltpu.VMEM((P, FEAT), jnp.float32),
        pltpu.VMEM((P,), jnp.int32),
        pltpu.VMEM((P,), jnp.int32),
        pltpu.VMEM((NSEG, FEAT), jnp.float32),
        pltpu.VMEM((NSEG, LANES), jnp.float32),
        pltpu.SemaphoreType.DMA,
        pltpu.SemaphoreType.DMA,
    ),
)(_sc_segsum_body)


BLK = 5000
NBLK = N_ROWS // BLK


def _main_body(parts_ref, cnts_ref, wt_ref, b_ref, x_ref, seg_ref, out_ref,
               wct_s, corr_s):
    @pl.when(pl.program_id(0) == 0)
    def _prep():
        sums = jnp.sum(parts_ref[...], axis=0)
        cnt = jnp.sum(cnts_ref[...], axis=0)
        cnt0 = cnt[:, 0:1]
        mean = sums / jnp.maximum(cnt0, 1.0)
        w1t = wt_ref[0:FEAT, :]
        w2t = wt_ref[FEAT:2 * FEAT, :]
        corr = b_ref[...] - jnp.dot(mean, w2t,
                                    preferred_element_type=jnp.float32)
        corr_s[...] = corr.astype(jnp.bfloat16)
        wct_s[...] = (w1t + w2t).astype(jnp.bfloat16)

    xb = x_ref[...].astype(jnp.bfloat16)
    acc = jnp.dot(xb, wct_s[...], preferred_element_type=jnp.float32)
    seg = seg_ref[...].reshape(1, BLK)
    onehot = (lax.broadcasted_iota(jnp.int32, (NSEG, BLK), 0)
              == jnp.broadcast_to(seg, (NSEG, BLK))).astype(jnp.bfloat16)
    acc = acc + lax.dot_general(onehot, corr_s[...],
                                (((0,), (0,)), ((), ())),
                                preferred_element_type=jnp.float32)
    out_ref[...] = acc


def _main(parts, cnts, wt, b2, x, seg3):
    return pl.pallas_call(
        _main_body,
        grid=(NBLK,),
        in_specs=[
            pl.BlockSpec((NW, NSEG, FEAT), lambda i: (0, 0, 0)),
            pl.BlockSpec((NW, NSEG, LANES), lambda i: (0, 0, 0)),
            pl.BlockSpec((2 * FEAT, FEAT), lambda i: (0, 0)),
            pl.BlockSpec((1, FEAT), lambda i: (0, 0)),
            pl.BlockSpec((BLK, FEAT), lambda i: (i, 0)),
            pl.BlockSpec((1, 1, BLK), lambda i: (i, 0, 0)),
        ],
        out_specs=pl.BlockSpec((BLK, FEAT), lambda i: (i, 0)),
        out_shape=jax.ShapeDtypeStruct((N_ROWS, FEAT), jnp.float32),
        scratch_shapes=[
            pltpu.VMEM((FEAT, FEAT), jnp.bfloat16),
            pltpu.VMEM((NSEG, FEAT), jnp.bfloat16),
        ],
    )(parts, cnts, wt, b2, x, seg3)


def kernel(x, batch, W, b):
    batch32 = batch.astype(jnp.int32)
    parts, cnts = _sc_segsum(x, batch32)
    wt = W.T
    b2 = b.reshape(1, FEAT)
    seg3 = batch32.reshape(NBLK, 1, BLK)
    return _main(parts, cnts, wt, b2, x, seg3)
